# baseline (speedup 1.0000x reference)
import jax
import jax.numpy as jnp
from jax import lax
from jax.experimental import pallas as pl
from jax.experimental.pallas import tpu as pltpu
from jax.experimental.pallas import tpu_sc as plsc

VOCAB = 100
HIDDEN = 1024
BATCH = 4096
SEQ = 20

NC, NS, L = 2, 16, 16
NW = NC * NS
NROWS = BATCH * SEQ
BPW = NROWS // NW
CR = 8
NCH = BPW // CR
NPAIR = NCH // 2
NJ = HIDDEN // L

_mesh = plsc.VectorSubcoreMesh(core_axis_name="c", subcore_axis_name="s")


@jax.jit
def _sc_gather(table, idx):
    @pl.kernel(
        out_type=jax.ShapeDtypeStruct((NROWS, HIDDEN), jnp.float32),
        mesh=_mesh,
        scratch_types=[
            pltpu.VMEM((VOCAB, HIDDEN), jnp.float32),
            pltpu.VMEM((BPW,), jnp.int32),
            pltpu.VMEM((CR, HIDDEN), jnp.float32),
            pltpu.VMEM((CR, HIDDEN), jnp.float32),
            pltpu.SemaphoreType.DMA,
            pltpu.SemaphoreType.DMA,
        ],
        compiler_params=pltpu.CompilerParams(needs_layout_passes=False),
    )
    def k(table_hbm, idx_hbm, out_hbm, table_v, idx_v, buf0, buf1, sem0, sem1):
        wid = lax.axis_index("s") * NC + lax.axis_index("c")
        base = wid * BPW
        pltpu.sync_copy(table_hbm, table_v)
        pltpu.sync_copy(idx_hbm.at[wid], idx_v)
        bufs = (buf0, buf1)
        sems = (sem0, sem1)
        iota = lax.iota(jnp.int32, L)

        def pair_body(q, carry):
            idx16 = idx_v[pl.ds(q * L, L)]
            for h in range(2):
                g = q * 2 + h
                @pl.when(q > 0)
                def _():
                    pltpu.make_async_copy(
                        bufs[h], out_hbm.at[pl.ds(base, CR)], sems[h]
                    ).wait()
                for rr in range(CR):
                    lane = h * CR + rr
                    rsplat = lax.gather(
                        idx16,
                        jnp.full((L, 1), lane, jnp.int32),
                        lax.GatherDimensionNumbers(
                            offset_dims=(),
                            collapsed_slice_dims=(0,),
                            start_index_map=(0,),
                        ),
                        (1,),
                        mode=lax.GatherScatterMode.PROMISE_IN_BOUNDS,
                    )
                    colv = iota
                    for j in range(NJ):
                        v = plsc.load_gather(table_v, [rsplat, colv])
                        bufs[h][rr, pl.ds(j * L, L)] = v
                        colv = colv + L
                pltpu.async_copy(
                    bufs[h], out_hbm.at[pl.ds(base + g * CR, CR)], sems[h]
                )
            return carry

        lax.fori_loop(0, NPAIR, pair_body, 0)
        for h in range(2):
            pltpu.make_async_copy(
                bufs[h], out_hbm.at[pl.ds(base, CR)], sems[h]
            ).wait()

    return k(table, idx)


def kernel(indices, word_embeddings):
    idx = indices.reshape(NW, BPW)
    out = _sc_gather(word_embeddings, idx)
    return out.reshape(BATCH, SEQ, HIDDEN)


# speedup vs baseline: 1.3053x; 1.3053x over previous
import jax
import jax.numpy as jnp
from jax import lax
from jax.experimental import pallas as pl
from jax.experimental.pallas import tpu as pltpu
from jax.experimental.pallas import tpu_sc as plsc

VOCAB = 100
HIDDEN = 1024
BATCH = 4096
SEQ = 20

NC, NS, L = 2, 16, 16
NW = NC * NS
NROWS = BATCH * SEQ
BPW = NROWS // NW
CR = 8
NCH = BPW // CR
NPAIR = NCH // 2
NJ = HIDDEN // L

_mesh = plsc.VectorSubcoreMesh(core_axis_name="c", subcore_axis_name="s")


@jax.jit
def _sc_gather(table, idx):
    @pl.kernel(
        out_type=jax.ShapeDtypeStruct((NROWS, HIDDEN), jnp.float32),
        mesh=_mesh,
        scratch_types=[
            pltpu.VMEM((VOCAB, HIDDEN), jnp.float32),
            pltpu.VMEM((BPW,), jnp.int32),
            pltpu.VMEM((CR, HIDDEN), jnp.float32),
            pltpu.VMEM((CR, HIDDEN), jnp.float32),
            pltpu.SemaphoreType.DMA,
            pltpu.SemaphoreType.DMA,
        ],
        compiler_params=pltpu.CompilerParams(needs_layout_passes=False),
    )
    def k(table_hbm, idx_hbm, out_hbm, table_v, idx_v, buf0, buf1, sem0, sem1):
        wid = lax.axis_index("s") * NC + lax.axis_index("c")
        base = wid * BPW
        pltpu.sync_copy(table_hbm, table_v)
        pltpu.sync_copy(idx_hbm.at[wid], idx_v)
        bufs = (buf0, buf1)
        sems = (sem0, sem1)
        iota = lax.iota(jnp.int32, L)

        def pair_body(q, carry):
            idx16 = idx_v[pl.ds(q * L, L)]
            for h in range(2):
                g = q * 2 + h
                @pl.when(q > 0)
                def _():
                    pltpu.make_async_copy(
                        bufs[h], out_hbm.at[pl.ds(base, CR)], sems[h]
                    ).wait()
                for rr in range(CR):
                    lane = h * CR + rr
                    rsplat = lax.gather(
                        idx16,
                        jnp.full((L, 1), lane, jnp.int32),
                        lax.GatherDimensionNumbers(
                            offset_dims=(),
                            collapsed_slice_dims=(0,),
                            start_index_map=(0,),
                        ),
                        (1,),
                        mode=lax.GatherScatterMode.PROMISE_IN_BOUNDS,
                    )
                    colv = iota
                    for jg in range(NJ // 4):
                        vs = []
                        for u in range(4):
                            vs.append(plsc.load_gather(table_v, [rsplat, colv]))
                            colv = colv + L
                        for u in range(4):
                            bufs[h][rr, pl.ds((jg * 4 + u) * L, L)] = vs[u]
                pltpu.async_copy(
                    bufs[h], out_hbm.at[pl.ds(base + g * CR, CR)], sems[h]
                )
            return carry

        lax.fori_loop(0, NPAIR, pair_body, 0)
        for h in range(2):
            pltpu.make_async_copy(
                bufs[h], out_hbm.at[pl.ds(base, CR)], sems[h]
            ).wait()

    return k(table, idx)


def kernel(indices, word_embeddings):
    idx = indices.reshape(NW, BPW)
    out = _sc_gather(word_embeddings, idx)
    return out.reshape(BATCH, SEQ, HIDDEN)


# speedup vs baseline: 1.3089x; 1.0028x over previous
import jax
import jax.numpy as jnp
from jax import lax
from jax.experimental import pallas as pl
from jax.experimental.pallas import tpu as pltpu
from jax.experimental.pallas import tpu_sc as plsc

VOCAB = 100
HIDDEN = 1024
BATCH = 4096
SEQ = 20

NC, NS, L = 2, 16, 16
NW = NC * NS
NROWS = BATCH * SEQ
BPW = NROWS // NW
CR = 8
NCH = BPW // CR
NPAIR = NCH // 2
NJ = HIDDEN // L

_mesh = plsc.VectorSubcoreMesh(core_axis_name="c", subcore_axis_name="s")


@jax.jit
def _sc_gather(table, idx):
    @pl.kernel(
        out_type=jax.ShapeDtypeStruct((NROWS, HIDDEN), jnp.float32),
        mesh=_mesh,
        scratch_types=[
            pltpu.VMEM((VOCAB, HIDDEN), jnp.float32),
            pltpu.VMEM((BPW,), jnp.int32),
            pltpu.VMEM((CR, HIDDEN), jnp.float32),
            pltpu.VMEM((CR, HIDDEN), jnp.float32),
            pltpu.SemaphoreType.DMA,
            pltpu.SemaphoreType.DMA,
        ],
        compiler_params=pltpu.CompilerParams(needs_layout_passes=False, use_tc_tiling_on_sc=True),
    )
    def k(table_hbm, idx_hbm, out_hbm, table_v, idx_v, buf0, buf1, sem0, sem1):
        wid = lax.axis_index("s") * NC + lax.axis_index("c")
        base = wid * BPW
        pltpu.sync_copy(table_hbm, table_v)
        pltpu.sync_copy(idx_hbm.at[wid], idx_v)
        bufs = (buf0, buf1)
        sems = (sem0, sem1)
        iota = lax.iota(jnp.int32, L)

        def pair_body(q, carry):
            idx16 = idx_v[pl.ds(q * L, L)]
            for h in range(2):
                g = q * 2 + h
                @pl.when(q > 0)
                def _():
                    pltpu.make_async_copy(
                        bufs[h], out_hbm.at[pl.ds(base, CR)], sems[h]
                    ).wait()
                for rr in range(CR):
                    lane = h * CR + rr
                    rsplat = lax.gather(
                        idx16,
                        jnp.full((L, 1), lane, jnp.int32),
                        lax.GatherDimensionNumbers(
                            offset_dims=(),
                            collapsed_slice_dims=(0,),
                            start_index_map=(0,),
                        ),
                        (1,),
                        mode=lax.GatherScatterMode.PROMISE_IN_BOUNDS,
                    )
                    colv = iota
                    for jg in range(NJ // 4):
                        vs = []
                        for u in range(4):
                            vs.append(plsc.load_gather(table_v, [rsplat, colv]))
                            colv = colv + L
                        for u in range(4):
                            bufs[h][rr, pl.ds((jg * 4 + u) * L, L)] = vs[u]
                pltpu.async_copy(
                    bufs[h], out_hbm.at[pl.ds(base + g * CR, CR)], sems[h]
                )
            return carry

        lax.fori_loop(0, NPAIR, pair_body, 0)
        for h in range(2):
            pltpu.make_async_copy(
                bufs[h], out_hbm.at[pl.ds(base, CR)], sems[h]
            ).wait()

    return k(table, idx)


def kernel(indices, word_embeddings):
    idx = indices.reshape(NW, BPW)
    out = _sc_gather(word_embeddings, idx)
    return out.reshape(BATCH, SEQ, HIDDEN)


# speedup vs baseline: 1.8786x; 1.4352x over previous
import jax
import jax.numpy as jnp
from jax import lax
from jax.experimental import pallas as pl
from jax.experimental.pallas import tpu as pltpu
from jax.experimental.pallas import tpu_sc as plsc

VOCAB = 100
HIDDEN = 1024
BATCH = 4096
SEQ = 20

NC, NS, L = 2, 16, 16
NW = NC * NS
NB = BATCH // NW
NSLOT = 4

_mesh = plsc.VectorSubcoreMesh(core_axis_name="c", subcore_axis_name="s")


@jax.jit
def _sc_gather(table, idx):
    @pl.kernel(
        out_type=jax.ShapeDtypeStruct((BATCH, SEQ, HIDDEN), jnp.float32),
        mesh=_mesh,
        scratch_types=[
            pltpu.VMEM((NB, SEQ), jnp.int32),
            pltpu.VMEM((SEQ, HIDDEN), jnp.float32),
            pltpu.VMEM((SEQ, HIDDEN), jnp.float32),
            pltpu.VMEM((SEQ, HIDDEN), jnp.float32),
            pltpu.VMEM((SEQ, HIDDEN), jnp.float32),
            pltpu.SemaphoreType.DMA,
            pltpu.SemaphoreType.DMA,
            pltpu.SemaphoreType.DMA,
            pltpu.SemaphoreType.DMA,
            pltpu.SemaphoreType.DMA,
            pltpu.SemaphoreType.DMA,
            pltpu.SemaphoreType.DMA,
            pltpu.SemaphoreType.DMA,
        ],
    )
    def k(table_hbm, idx_hbm, out_hbm,
          idx_v, b0, b1, b2, b3,
          g0, g1, g2, g3, w0, w1, w2, w3):
        wid = lax.axis_index("s") * NC + lax.axis_index("c")
        base = wid * NB
        pltpu.sync_copy(idx_hbm.at[wid], idx_v)
        bufs = (b0, b1, b2, b3)
        gsems = (g0, g1, g2, g3)
        wsems = (w0, w1, w2, w3)

        pltpu.async_copy(table_hbm.at[idx_v.at[0]], bufs[0], gsems[0])
        pltpu.async_copy(table_hbm.at[idx_v.at[1]], bufs[1], gsems[1])

        def step(bb, s):
            sn = (s + 2) % NSLOT
            @pl.when(bb + 2 < NB)
            def _():
                @pl.when(bb >= 2)
                def _():
                    pltpu.make_async_copy(
                        bufs[sn], out_hbm.at[base + bb - 2], wsems[sn]
                    ).wait()
                pltpu.async_copy(
                    table_hbm.at[idx_v.at[bb + 2]], bufs[sn], gsems[sn]
                )
            pltpu.make_async_copy(
                table_hbm.at[idx_v.at[bb]], bufs[s], gsems[s]
            ).wait()
            pltpu.async_copy(bufs[s], out_hbm.at[base + bb], wsems[s])

        def quad(q, carry):
            for s in range(NSLOT):
                step(q * NSLOT + s, s)
            return carry

        lax.fori_loop(0, NB // NSLOT, quad, 0)
        for s in range(NSLOT):
            pltpu.make_async_copy(
                bufs[s], out_hbm.at[base], wsems[s]
            ).wait()

    return k(table, idx)


def kernel(indices, word_embeddings):
    idx = indices.reshape(NW, NB, SEQ)
    return _sc_gather(word_embeddings, idx)
